# baseline (speedup 1.0000x reference)
import functools

import jax
import jax.numpy as jnp
from jax import lax
from jax.experimental import pallas as pl
from jax.experimental.pallas import tpu as pltpu
from jax.experimental.pallas import tpu_sc as plsc

VOCAB = 100000
EMBED_DIM = 16
BATCH = 1024
HIST = 200

_C0 = 104
_C1 = HIST - _C0


def _sc_gather_mean(x, emb, row0, nrows):
    info = plsc.get_sparse_core_info()
    nc, ns = info.num_cores, info.num_subcores
    nw = nc * ns
    bpw = nrows // nw

    mesh = plsc.VectorSubcoreMesh(core_axis_name="c", subcore_axis_name="s")

    @functools.partial(
        pl.kernel,
        mesh=mesh,
        out_type=jax.ShapeDtypeStruct((nrows, EMBED_DIM), jnp.float32),
        scratch_types=[
            pltpu.VMEM((bpw * HIST,), jnp.int32),
            pltpu.VMEM((8, HIST, EMBED_DIM), jnp.float32),
            pltpu.VMEM((bpw, EMBED_DIM), jnp.float32),
            pltpu.SemaphoreType.DMA((8,)),
        ],
        compiler_params=pltpu.CompilerParams(use_tc_tiling_on_sc=False),
    )
    def sc_kernel(x_hbm, emb_hbm, h_hbm, xv, rows, hv, sems):
        wid = lax.axis_index("s") * nc + lax.axis_index("c")
        base = wid * bpw
        pltpu.sync_copy(x_hbm.at[pl.ds((row0 + base) * HIST, bpw * HIST)], xv)

        def gather_row(r, s):
            pltpu.make_async_copy(
                emb_hbm.at[xv.at[pl.ds(r * HIST, _C0)]],
                rows.at[s, pl.ds(0, _C0)], sems.at[s]).start()
            pltpu.make_async_copy(
                emb_hbm.at[xv.at[pl.ds(r * HIST + _C0, _C1)]],
                rows.at[s, pl.ds(_C0, _C1)], sems.at[s]).start()

        def wait_row(r, s):
            pltpu.make_async_copy(
                emb_hbm.at[xv.at[pl.ds(r * HIST, _C0)]],
                rows.at[s, pl.ds(0, _C0)], sems.at[s]).wait()
            pltpu.make_async_copy(
                emb_hbm.at[xv.at[pl.ds(r * HIST + _C0, _C1)]],
                rows.at[s, pl.ds(_C0, _C1)], sems.at[s]).wait()

        for r in range(7):
            gather_row(r, r)

        def row_body(i, carry):
            slot = lax.rem(i, 8)

            @pl.when(i + 7 < bpw)
            def _prefetch():
                gather_row(i + 7, lax.rem(i + 7, 8))

            wait_row(i, slot)

            def red(j, acc):
                a = acc
                for u in range(8):
                    a = a + rows[slot, 8 * j + u]
                return a

            acc = lax.fori_loop(0, HIST // 8, red, jnp.zeros((EMBED_DIM,), jnp.float32))
            hv[i] = acc * jnp.float32(1.0 / HIST)
            return carry

        lax.fori_loop(0, bpw, row_body, 0)
        pltpu.sync_copy(hv, h_hbm.at[pl.ds(base, bpw)])

    return sc_kernel(x, emb)


def _tc_matmul(h, W, b2d):
    mb = 8
    nsteps = BATCH // mb
    nbuf = 4

    def body(h_hbm, w_hbm, b_hbm, o_ref, buf, hbuf, wbuf, bbuf, sems, insem):
        j = pl.program_id(0)
        slot = lax.rem(j, nbuf)

        @pl.when(j == 0)
        def _load_inputs():
            pltpu.make_async_copy(h_hbm, hbuf, insem).start()
            pltpu.make_async_copy(b_hbm, bbuf, insem).start()
            pltpu.make_async_copy(w_hbm, wbuf, insem).start()
            pltpu.make_async_copy(h_hbm, hbuf, insem).wait()
            pltpu.make_async_copy(b_hbm, bbuf, insem).wait()
            pltpu.make_async_copy(w_hbm, wbuf, insem).wait()

        @pl.when(j >= nbuf)
        def _wait_old():
            pltpu.make_async_copy(
                buf.at[slot], o_ref.at[pl.ds((j - nbuf) * mb, mb)], sems.at[slot]
            ).wait()

        buf[slot] = (
            jnp.dot(
                hbuf[pl.ds(j * mb, mb)], wbuf[...],
                preferred_element_type=jnp.float32,
            )
            + bbuf[...]
        )
        pltpu.make_async_copy(
            buf.at[slot], o_ref.at[pl.ds(j * mb, mb)], sems.at[slot]
        ).start()

        @pl.when(j == nsteps - 1)
        def _drain():
            for k in range(1, nbuf + 1):
                s = lax.rem(j - nbuf + k + nbuf, nbuf)
                pltpu.make_async_copy(
                    buf.at[s],
                    o_ref.at[pl.ds((j - nbuf + k) * mb, mb)],
                    sems.at[s],
                ).wait()

    return pl.pallas_call(
        body,
        grid=(nsteps,),
        in_specs=[
            pl.BlockSpec(memory_space=pl.ANY),
            pl.BlockSpec(memory_space=pl.ANY),
            pl.BlockSpec(memory_space=pl.ANY),
        ],
        out_specs=pl.BlockSpec(memory_space=pl.ANY),
        out_shape=jax.ShapeDtypeStruct((BATCH, VOCAB), jnp.float32),
        scratch_shapes=[
            pltpu.VMEM((nbuf, mb, VOCAB), jnp.float32),
            pltpu.VMEM((BATCH, EMBED_DIM), jnp.float32),
            pltpu.VMEM((EMBED_DIM, VOCAB), jnp.float32),
            pltpu.VMEM((1, VOCAB), jnp.float32),
            pltpu.SemaphoreType.DMA((nbuf,)),
            pltpu.SemaphoreType.DMA,
        ],
    )(h, W, b2d)


def kernel(x, emb, W, b):
    h = _sc_gather_mean(x.reshape(BATCH * HIST), emb, 0, BATCH)
    return _tc_matmul(h, W, b.reshape(1, VOCAB))


# speedup vs baseline: 1.0013x; 1.0013x over previous
import functools

import jax
import jax.numpy as jnp
from jax import lax
from jax.experimental import pallas as pl
from jax.experimental.pallas import tpu as pltpu
from jax.experimental.pallas import tpu_sc as plsc

VOCAB = 100000
EMBED_DIM = 16
BATCH = 1024
HIST = 200

_C0 = 128
_C1 = HIST - _C0


def _sc_gather_mean(x, emb, row0, nrows):
    info = plsc.get_sparse_core_info()
    nc, ns = info.num_cores, info.num_subcores
    nw = nc * ns
    bpw = nrows // nw

    mesh = plsc.VectorSubcoreMesh(core_axis_name="c", subcore_axis_name="s")

    @functools.partial(
        pl.kernel,
        mesh=mesh,
        out_type=jax.ShapeDtypeStruct((nrows, EMBED_DIM), jnp.float32),
        scratch_types=[
            pltpu.VMEM((bpw * HIST,), jnp.int32),
            pltpu.VMEM((6, HIST, EMBED_DIM), jnp.float32),
            pltpu.VMEM((bpw, EMBED_DIM), jnp.float32),
            pltpu.SemaphoreType.DMA((6,)),
        ],
        compiler_params=pltpu.CompilerParams(use_tc_tiling_on_sc=False),
    )
    def sc_kernel(x_hbm, emb_hbm, h_hbm, xv, rows, hv, sems):
        wid = lax.axis_index("s") * nc + lax.axis_index("c")
        base = wid * bpw
        pltpu.sync_copy(x_hbm.at[pl.ds((row0 + base) * HIST, bpw * HIST)], xv)

        def gather_row(r, s):
            pltpu.make_async_copy(
                emb_hbm.at[xv.at[pl.ds(r * HIST, _C0)]],
                rows.at[s, pl.ds(0, _C0)], sems.at[s]).start()
            pltpu.make_async_copy(
                emb_hbm.at[xv.at[pl.ds(r * HIST + _C0, _C1)]],
                rows.at[s, pl.ds(_C0, _C1)], sems.at[s]).start()

        def wait_row(r, s):
            pltpu.make_async_copy(
                emb_hbm.at[xv.at[pl.ds(r * HIST, _C0)]],
                rows.at[s, pl.ds(0, _C0)], sems.at[s]).wait()
            pltpu.make_async_copy(
                emb_hbm.at[xv.at[pl.ds(r * HIST + _C0, _C1)]],
                rows.at[s, pl.ds(_C0, _C1)], sems.at[s]).wait()

        for r in range(5):
            gather_row(r, r)

        def row_body(i, carry):
            slot = lax.rem(i, 6)

            @pl.when(i + 5 < bpw)
            def _prefetch():
                gather_row(i + 5, lax.rem(i + 5, 6))

            wait_row(i, slot)

            def red(j, acc):
                a = acc
                for u in range(8):
                    a = a + rows[slot, 8 * j + u]
                return a

            acc = lax.fori_loop(0, HIST // 8, red, jnp.zeros((EMBED_DIM,), jnp.float32))
            hv[i] = acc * jnp.float32(1.0 / HIST)
            return carry

        lax.fori_loop(0, bpw, row_body, 0)
        pltpu.sync_copy(hv, h_hbm.at[pl.ds(base, bpw)])

    return sc_kernel(x, emb)


def _tc_matmul(h, W, b2d):
    mb = 8
    nsteps = BATCH // mb
    nbuf = 4

    def body(h_ref, w_ref, b_ref, o_ref, buf, sems):
        j = pl.program_id(0)
        slot = lax.rem(j, nbuf)

        @pl.when(j >= nbuf)
        def _wait_old():
            pltpu.make_async_copy(
                buf.at[slot], o_ref.at[pl.ds((j - nbuf) * mb, mb)], sems.at[slot]
            ).wait()

        buf[slot] = (
            jnp.dot(h_ref[...], w_ref[...], preferred_element_type=jnp.float32)
            + b_ref[...]
        )
        pltpu.make_async_copy(
            buf.at[slot], o_ref.at[pl.ds(j * mb, mb)], sems.at[slot]
        ).start()

        @pl.when(j == nsteps - 1)
        def _drain():
            for k in range(1, nbuf + 1):
                s = lax.rem(j - nbuf + k + nbuf, nbuf)
                pltpu.make_async_copy(
                    buf.at[s],
                    o_ref.at[pl.ds((j - nbuf + k) * mb, mb)],
                    sems.at[s],
                ).wait()

    return pl.pallas_call(
        body,
        grid=(nsteps,),
        in_specs=[
            pl.BlockSpec((mb, EMBED_DIM), lambda j: (j, 0)),
            pl.BlockSpec((EMBED_DIM, VOCAB), lambda j: (0, 0)),
            pl.BlockSpec((1, VOCAB), lambda j: (0, 0)),
        ],
        out_specs=pl.BlockSpec(memory_space=pl.ANY),
        out_shape=jax.ShapeDtypeStruct((BATCH, VOCAB), jnp.float32),
        scratch_shapes=[
            pltpu.VMEM((nbuf, mb, VOCAB), jnp.float32),
            pltpu.SemaphoreType.DMA((nbuf,)),
        ],
    )(h, W, b2d)


def kernel(x, emb, W, b):
    h = _sc_gather_mean(x.reshape(BATCH * HIST), emb, 0, BATCH)
    return _tc_matmul(h, W, b.reshape(1, VOCAB))
